# baseline (speedup 1.0000x reference)
import jax
import jax.numpy as jnp
from jax import lax
from jax.experimental import pallas as pl
from jax.experimental.pallas import tpu as pltpu
from jax.experimental.pallas import tpu_sc as plsc

B = 16
L = 4096
D = 128
NC = 2
NS = 16
HALF = L // 2
REP = 32
REP_LOG2 = 5


def _body(x_hbm, table_hbm, out_hbm, xrow_v, table_v, const_v, sem):
    wid = lax.axis_index("s") * NC + lax.axis_index("c")
    b = wid >> 1
    h = wid & 1

    cp_t = pltpu.async_copy(table_hbm, table_v, sem)
    cp_x = pltpu.async_copy(x_hbm.at[b], xrow_v, sem)
    cp_t.wait()
    cp_x.wait()

    rowvecs = [
        [table_v[r, pl.ds(j * 16, 16)] for j in range(D // 16)]
        for r in range(3)
    ]
    lanes = lax.iota(jnp.int32, 16)
    big = jnp.full((16,), L, jnp.int32)

    def fused_step(i, carry):
        a4, a2 = carry
        for r in range(3):
            for j in range(D // 16):
                const_v[r * REP + i, pl.ds(j * 16, 16)] = rowvecs[r][j]
        for u in range(L // 16 // REP):
            c = i * (L // 16 // REP) + u
            v = xrow_v[pl.ds(c * 16, 16)]
            pos = lanes + c * 16
            a4 = jnp.minimum(a4, jnp.where(v == 4, pos, L))
            a2 = jnp.minimum(a2, jnp.where(v == 2, pos, L))
        return a4, a2

    a4, a2 = lax.fori_loop(0, REP, fused_step, (big, big))
    sep = jnp.min(a4)
    eos = jnp.min(a2)

    t0 = h * HALF
    t1 = t0 + HALF
    s = jnp.minimum(jnp.maximum(sep, t0), t1)
    e = jnp.minimum(jnp.maximum(eos, s), t1)

    def copy_run(rid, a0, a1):
        ln = a1 - a0

        def chunk(i, _):
            pltpu.async_copy(
                const_v.at[pl.ds(rid * REP, REP)],
                out_hbm.at[b, pl.ds(a0 + (i << REP_LOG2), REP)],
                sem,
            )
            return 0

        lax.fori_loop(0, ln >> REP_LOG2, chunk, 0)
        rem = ln & (REP - 1)
        base = a0 + (ln - rem)
        for bp in range(REP_LOG2 - 1, -1, -1):
            k = 1 << bp
            off = base + ((rem >> (bp + 1)) << (bp + 1))

            @pl.when(((rem >> bp) & 1) == 1)
            def _(k=k, off=off):
                pltpu.async_copy(
                    const_v.at[pl.ds(rid * REP, k)],
                    out_hbm.at[b, pl.ds(off, k)],
                    sem,
                )

    copy_run(1, t0, s)
    copy_run(2, s, e)
    copy_run(0, e, t1)

    span = out_hbm.at[b, pl.ds(t0, HALF)]
    pltpu.make_async_copy(span, span, sem).wait()


_sc_call = pl.kernel(
    _body,
    out_type=jax.ShapeDtypeStruct((B, L, D), jnp.float32),
    mesh=plsc.VectorSubcoreMesh(core_axis_name="c", subcore_axis_name="s"),
    compiler_params=pltpu.CompilerParams(
        use_tc_tiling_on_sc=False, needs_layout_passes=False
    ),
    scratch_types=[
        pltpu.VMEM((L,), jnp.int32),
        pltpu.VMEM((3, D), jnp.float32),
        pltpu.VMEM((3 * REP, D), jnp.float32),
        pltpu.SemaphoreType.DMA,
    ],
)


@jax.jit
def kernel(x, seg_table):
    return _sc_call(x, seg_table)


# speedup vs baseline: 1.0168x; 1.0168x over previous
import jax
import jax.numpy as jnp
from jax import lax
from jax.experimental import pallas as pl
from jax.experimental.pallas import tpu as pltpu
from jax.experimental.pallas import tpu_sc as plsc

B = 16
L = 4096
D = 128
NC = 2
NS = 16
HALF = L // 2
REP = 64
REP_LOG2 = 6


def _body(x_hbm, table_hbm, out_hbm, xrow_v, table_v, const_v, sem):
    wid = lax.axis_index("s") * NC + lax.axis_index("c")
    b = wid >> 1
    h = wid & 1

    cp_t = pltpu.async_copy(table_hbm, table_v, sem)
    cp_x = pltpu.async_copy(x_hbm.at[b], xrow_v, sem)
    cp_t.wait()
    cp_x.wait()

    rowvecs = [
        [table_v[r, pl.ds(j * 16, 16)] for j in range(D // 16)]
        for r in range(3)
    ]
    lanes = lax.iota(jnp.int32, 16)
    big = jnp.full((16,), L, jnp.int32)

    def fused_step(i, carry):
        a4, a2 = carry
        for r in range(3):
            for j in range(D // 16):
                const_v[r * REP + i, pl.ds(j * 16, 16)] = rowvecs[r][j]
        for u in range(L // 16 // REP):
            c = i * (L // 16 // REP) + u
            v = xrow_v[pl.ds(c * 16, 16)]
            pos = lanes + c * 16
            a4 = jnp.minimum(a4, jnp.where(v == 4, pos, L))
            a2 = jnp.minimum(a2, jnp.where(v == 2, pos, L))
        return a4, a2

    a4, a2 = lax.fori_loop(0, REP, fused_step, (big, big))
    sep = jnp.min(a4)
    eos = jnp.min(a2)

    t0 = h * HALF
    t1 = t0 + HALF
    s = jnp.minimum(jnp.maximum(sep, t0), t1)
    e = jnp.minimum(jnp.maximum(eos, s), t1)

    def copy_run(rid, a0, a1):
        ln = a1 - a0

        def chunk(i, _):
            pltpu.async_copy(
                const_v.at[pl.ds(rid * REP, REP)],
                out_hbm.at[b, pl.ds(a0 + (i << REP_LOG2), REP)],
                sem,
            )
            return 0

        lax.fori_loop(0, ln >> REP_LOG2, chunk, 0)
        rem = ln & (REP - 1)
        base = a0 + (ln - rem)
        for bp in range(REP_LOG2 - 1, -1, -1):
            k = 1 << bp
            off = base + ((rem >> (bp + 1)) << (bp + 1))

            @pl.when(((rem >> bp) & 1) == 1)
            def _(k=k, off=off):
                pltpu.async_copy(
                    const_v.at[pl.ds(rid * REP, k)],
                    out_hbm.at[b, pl.ds(off, k)],
                    sem,
                )

    copy_run(1, t0, s)
    copy_run(2, s, e)
    copy_run(0, e, t1)

    span = out_hbm.at[b, pl.ds(t0, HALF)]
    pltpu.make_async_copy(span, span, sem).wait()


_sc_call = pl.kernel(
    _body,
    out_type=jax.ShapeDtypeStruct((B, L, D), jnp.float32),
    mesh=plsc.VectorSubcoreMesh(core_axis_name="c", subcore_axis_name="s"),
    compiler_params=pltpu.CompilerParams(
        use_tc_tiling_on_sc=False, needs_layout_passes=False, disable_bounds_checks=True
    ),
    scratch_types=[
        pltpu.VMEM((L,), jnp.int32),
        pltpu.VMEM((3, D), jnp.float32),
        pltpu.VMEM((3 * REP, D), jnp.float32),
        pltpu.SemaphoreType.DMA,
    ],
)


@jax.jit
def kernel(x, seg_table):
    return _sc_call(x, seg_table)
